# speedup vs baseline: 1.9367x; 1.9367x over previous
import functools

import jax
import jax.numpy as jnp
from jax import lax
from jax.experimental import pallas as pl
from jax.experimental.pallas import tpu as pltpu
from jax.experimental.pallas import tpu_sc as plsc

MAXDEG = 64
N = 10000
E = 320000
D = 128
NC, NS = 2, 16
HPAD = 10240
SLICE = HPAD // NS
CHUNK = 125
ROWS = E // CHUNK // NS


def _hist_body(edges, hist_out, idx_v, ones_v, zeros_v, hist_sh):
    c = lax.axis_index("c")
    s = lax.axis_index("s")

    for i in range(8):
        ones_v[pl.ds(i * 16, 16)] = jnp.ones((16,), jnp.int32)
    zero16 = jnp.zeros((16,), jnp.int32)

    def zbody(i, carry):
        zeros_v[pl.ds(i * 16, 16)] = zero16
        return carry

    lax.fori_loop(0, SLICE // 16, zbody, 0)
    pltpu.sync_copy(zeros_v, hist_sh.at[pl.ds(s * SLICE, SLICE)])
    pltpu.sync_copy(edges.at[c, pl.ds(s * ROWS, ROWS)], idx_v)
    plsc.subcore_barrier()

    def sbody(j, carry):
        pltpu.sync_copy(ones_v.at[pl.ds(0, CHUNK)], hist_sh.at[idx_v.at[j]],
                        add=True)
        return carry

    lax.fori_loop(0, ROWS, sbody, 0)
    plsc.subcore_barrier()
    pltpu.sync_copy(hist_sh.at[pl.ds(s * SLICE, SLICE)],
                    hist_out.at[c, pl.ds(s * SLICE, SLICE)])


_hist_call = pl.kernel(
    _hist_body,
    out_type=jax.ShapeDtypeStruct((NC, HPAD), jnp.int32),
    mesh=plsc.VectorSubcoreMesh(core_axis_name="c", subcore_axis_name="s",
                                num_cores=NC, num_subcores=NS),
    scratch_types=[
        pltpu.VMEM((ROWS, CHUNK), jnp.int32),
        pltpu.VMEM((128,), jnp.int32),
        pltpu.VMEM((SLICE,), jnp.int32),
        pltpu.VMEM_SHARED((HPAD,), jnp.int32),
    ],
)

BLK = 1000
NBLK = N // BLK


def _encode_body(x_ref, din_ref, dout_ref, zin_ref, zout_ref, o_ref):
    din = jnp.minimum(din_ref[0], MAXDEG - 1)
    dout = jnp.minimum(dout_ref[0], MAXDEG - 1)
    iota = lax.broadcasted_iota(jnp.int32, (MAXDEG, BLK), 0)
    oh_in = (jnp.broadcast_to(din, (MAXDEG, BLK)) == iota).astype(jnp.float32)
    oh_out = (jnp.broadcast_to(dout, (MAXDEG, BLK)) == iota).astype(jnp.float32)
    dn = (((0,), (0,)), ((), ()))
    acc = x_ref[...]
    acc += lax.dot_general(oh_in, zin_ref[...], dn,
                           preferred_element_type=jnp.float32)
    acc += lax.dot_general(oh_out, zout_ref[...], dn,
                           preferred_element_type=jnp.float32)
    o_ref[...] = acc


_encode_call = pl.pallas_call(
    _encode_body,
    out_shape=jax.ShapeDtypeStruct((N, D), jnp.float32),
    grid=(NBLK,),
    in_specs=[
        pl.BlockSpec((BLK, D), lambda i: (i, 0)),
        pl.BlockSpec((1, 1, BLK), lambda i: (i, 0, 0)),
        pl.BlockSpec((1, 1, BLK), lambda i: (i, 0, 0)),
        pl.BlockSpec((MAXDEG, D), lambda i: (0, 0)),
        pl.BlockSpec((MAXDEG, D), lambda i: (0, 0)),
    ],
    out_specs=pl.BlockSpec((BLK, D), lambda i: (i, 0)),
)


def kernel(x, edge_index, z_in, z_out):
    e = edge_index.astype(jnp.int32).reshape(2, E // CHUNK, CHUNK)
    hist = _hist_call(e)
    d_out_deg = hist[0, :N].reshape(NBLK, 1, BLK)
    d_in_deg = hist[1, :N].reshape(NBLK, 1, BLK)
    return _encode_call(x, d_in_deg, d_out_deg, z_in, z_out)


# speedup vs baseline: 2.0950x; 1.0817x over previous
import functools

import jax
import jax.numpy as jnp
from jax import lax
from jax.experimental import pallas as pl
from jax.experimental.pallas import tpu as pltpu
from jax.experimental.pallas import tpu_sc as plsc

MAXDEG = 64
N = 10000
E = 320000
D = 128
NC, NS = 2, 16
HPAD = 10240
SLICE = HPAD // NS
EPAD = 327680
EPT = EPAD // NS
SENT = 10200


def _hist_body(edges, hist_out, idx_v, ones_v, zeros_v, hist_sh):
    c = lax.axis_index("c")
    s = lax.axis_index("s")

    one16 = jnp.ones((16,), jnp.int32)
    zero16 = jnp.zeros((16,), jnp.int32)

    def obody(i, carry):
        ones_v[pl.ds(i * 16, 16)] = one16
        return carry

    lax.fori_loop(0, EPT // 16, obody, 0)

    def zbody(i, carry):
        zeros_v[pl.ds(i * 16, 16)] = zero16
        return carry

    lax.fori_loop(0, SLICE // 16, zbody, 0)
    pltpu.sync_copy(zeros_v, hist_sh.at[pl.ds(s * SLICE, SLICE)])
    pltpu.sync_copy(edges.at[c, 0, pl.ds(s * EPT, EPT)], idx_v)
    plsc.subcore_barrier()
    pltpu.sync_copy(ones_v, hist_sh.at[idx_v], add=True)
    plsc.subcore_barrier()
    pltpu.sync_copy(hist_sh.at[pl.ds(s * SLICE, SLICE)],
                    hist_out.at[c, pl.ds(s * SLICE, SLICE)])


_hist_call = pl.kernel(
    _hist_body,
    out_type=jax.ShapeDtypeStruct((NC, HPAD), jnp.int32),
    mesh=plsc.VectorSubcoreMesh(core_axis_name="c", subcore_axis_name="s",
                                num_cores=NC, num_subcores=NS),
    scratch_types=[
        pltpu.VMEM((EPT,), jnp.int32),
        pltpu.VMEM((EPT,), jnp.int32),
        pltpu.VMEM((SLICE,), jnp.int32),
        pltpu.VMEM_SHARED((HPAD,), jnp.int32),
    ],
)

BLK = 1000
NBLK = N // BLK


def _encode_body(x_ref, h_ref, zin_ref, zout_ref, o_ref):
    dout = jnp.minimum(h_ref[0, 0], MAXDEG - 1)
    din = jnp.minimum(h_ref[1, 0], MAXDEG - 1)
    iota = lax.broadcasted_iota(jnp.int32, (MAXDEG, BLK), 0)
    oh_in = (jnp.broadcast_to(din, (MAXDEG, BLK)) == iota).astype(jnp.float32)
    oh_out = (jnp.broadcast_to(dout, (MAXDEG, BLK)) == iota).astype(jnp.float32)
    dn = (((0,), (0,)), ((), ()))
    acc = x_ref[...]
    acc += lax.dot_general(oh_in, zin_ref[...], dn,
                           preferred_element_type=jnp.float32)
    acc += lax.dot_general(oh_out, zout_ref[...], dn,
                           preferred_element_type=jnp.float32)
    o_ref[...] = acc


_encode_call = pl.pallas_call(
    _encode_body,
    out_shape=jax.ShapeDtypeStruct((N, D), jnp.float32),
    grid=(NBLK,),
    in_specs=[
        pl.BlockSpec((BLK, D), lambda i: (i, 0)),
        pl.BlockSpec((NC, 1, 1, BLK), lambda i: (0, i, 0, 0)),
        pl.BlockSpec((MAXDEG, D), lambda i: (0, 0)),
        pl.BlockSpec((MAXDEG, D), lambda i: (0, 0)),
    ],
    out_specs=pl.BlockSpec((BLK, D), lambda i: (i, 0)),
)


def kernel(x, edge_index, z_in, z_out):
    e = jnp.pad(edge_index.astype(jnp.int32), ((0, 0), (0, EPAD - E)),
                constant_values=SENT).reshape(NC, 1, EPAD)
    hist = _hist_call(e)
    d = hist[:, :N].reshape(NC, NBLK, 1, BLK)
    return _encode_call(x, d, z_in, z_out)


# speedup vs baseline: 2.1103x; 1.0073x over previous
import functools

import jax
import jax.numpy as jnp
from jax import lax
from jax.experimental import pallas as pl
from jax.experimental.pallas import tpu as pltpu
from jax.experimental.pallas import tpu_sc as plsc

MAXDEG = 64
N = 10000
E = 320000
D = 128
NC, NS = 2, 16
HPAD = 10240
SLICE = HPAD // NS
EPAD = 327680
EPT = EPAD // NS
SENT = 10200
BLK = 1000
NBLK = N // BLK


def _hist_body(edges, hist_out, idx_v, ones_v, zeros_v, buf_v, hist_sh):
    c = lax.axis_index("c")
    s = lax.axis_index("s")

    one16 = jnp.ones((16,), jnp.int32)
    zero16 = jnp.zeros((16,), jnp.int32)

    def obody(i, carry):
        ones_v[pl.ds(i * 16, 16)] = one16
        return carry

    lax.fori_loop(0, EPT // 16, obody, 0)

    def zbody(i, carry):
        zeros_v[pl.ds(i * 16, 16)] = zero16
        return carry

    lax.fori_loop(0, SLICE // 16, zbody, 0)
    pltpu.sync_copy(zeros_v, hist_sh.at[pl.ds(s * SLICE, SLICE)])
    pltpu.sync_copy(edges.at[c, 0, pl.ds(s * EPT, EPT)], idx_v)
    plsc.subcore_barrier()
    pltpu.sync_copy(ones_v, hist_sh.at[idx_v], add=True)
    plsc.subcore_barrier()

    @pl.when(s < NBLK)
    def _():
        pltpu.sync_copy(hist_sh.at[pl.ds(s * BLK, 1024)], buf_v)
        pltpu.sync_copy(buf_v, hist_out.at[c, s, 0])


_hist_call = pl.kernel(
    _hist_body,
    out_type=jax.ShapeDtypeStruct((NC, NBLK, 1, 1024), jnp.int32),
    mesh=plsc.VectorSubcoreMesh(core_axis_name="c", subcore_axis_name="s",
                                num_cores=NC, num_subcores=NS),
    scratch_types=[
        pltpu.VMEM((EPT,), jnp.int32),
        pltpu.VMEM((EPT,), jnp.int32),
        pltpu.VMEM((SLICE,), jnp.int32),
        pltpu.VMEM((1024,), jnp.int32),
        pltpu.VMEM_SHARED((HPAD,), jnp.int32),
    ],
)

def _encode_body(x_ref, h_ref, zin_ref, zout_ref, o_ref):
    dout = jnp.minimum(h_ref[0, 0][:, :BLK], MAXDEG - 1)
    din = jnp.minimum(h_ref[1, 0][:, :BLK], MAXDEG - 1)
    iota = lax.broadcasted_iota(jnp.int32, (MAXDEG, BLK), 0)
    oh_in = (jnp.broadcast_to(din, (MAXDEG, BLK)) == iota).astype(jnp.float32)
    oh_out = (jnp.broadcast_to(dout, (MAXDEG, BLK)) == iota).astype(jnp.float32)
    dn = (((0,), (0,)), ((), ()))
    acc = x_ref[...]
    acc += lax.dot_general(oh_in, zin_ref[...], dn,
                           preferred_element_type=jnp.float32)
    acc += lax.dot_general(oh_out, zout_ref[...], dn,
                           preferred_element_type=jnp.float32)
    o_ref[...] = acc


_encode_call = pl.pallas_call(
    _encode_body,
    out_shape=jax.ShapeDtypeStruct((N, D), jnp.float32),
    grid=(NBLK,),
    in_specs=[
        pl.BlockSpec((BLK, D), lambda i: (i, 0)),
        pl.BlockSpec((NC, 1, 1, 1024), lambda i: (0, i, 0, 0)),
        pl.BlockSpec((MAXDEG, D), lambda i: (0, 0)),
        pl.BlockSpec((MAXDEG, D), lambda i: (0, 0)),
    ],
    out_specs=pl.BlockSpec((BLK, D), lambda i: (i, 0)),
)


def kernel(x, edge_index, z_in, z_out):
    e = jnp.pad(edge_index.astype(jnp.int32).reshape(NC, 1, E),
                ((0, 0), (0, 0), (0, EPAD - E)), constant_values=SENT)
    hist = _hist_call(e)
    return _encode_call(x, hist, z_in, z_out)


# speedup vs baseline: 2.5405x; 1.2039x over previous
import functools

import jax
import jax.numpy as jnp
from jax import lax
from jax.experimental import pallas as pl
from jax.experimental.pallas import tpu as pltpu
from jax.experimental.pallas import tpu_sc as plsc

MAXDEG = 64
N = 10000
E = 320000
D = 128
NC, NS = 2, 16
HPAD = 10240
SLICE = HPAD // NS
CH = 19968
TAIL = E - CH * NS
BLK = 1000
NBLK = N // BLK


def _hist_body(edges, hist_out, idx_v, tail_v, ones_v, zeros_v, buf_v,
               hist_sh):
    c = lax.axis_index("c")
    s = lax.axis_index("s")

    one16 = jnp.ones((16,), jnp.int32)
    zero16 = jnp.zeros((16,), jnp.int32)

    def obody(i, carry):
        ones_v[pl.ds(i * 16, 16)] = one16
        return carry

    lax.fori_loop(0, CH // 16, obody, 0)

    def zbody(i, carry):
        zeros_v[pl.ds(i * 16, 16)] = zero16
        return carry

    lax.fori_loop(0, SLICE // 16, zbody, 0)
    pltpu.sync_copy(zeros_v, hist_sh.at[pl.ds(s * SLICE, SLICE)])
    pltpu.sync_copy(edges.at[c, pl.ds(s * CH, CH)], idx_v)

    @pl.when(s == 0)
    def _():
        pltpu.sync_copy(edges.at[c, pl.ds(CH * NS, TAIL)], tail_v)

    plsc.subcore_barrier()
    pltpu.sync_copy(ones_v, hist_sh.at[idx_v], add=True)

    @pl.when(s == 0)
    def _():
        pltpu.sync_copy(ones_v.at[pl.ds(0, TAIL)], hist_sh.at[tail_v],
                        add=True)

    plsc.subcore_barrier()

    @pl.when(s < NBLK)
    def _():
        pltpu.sync_copy(hist_sh.at[pl.ds(s * BLK, 1024)], buf_v)
        pltpu.sync_copy(buf_v, hist_out.at[c, s, 0])


_hist_call = pl.kernel(
    _hist_body,
    out_type=jax.ShapeDtypeStruct((NC, NBLK, 1, 1024), jnp.int32),
    mesh=plsc.VectorSubcoreMesh(core_axis_name="c", subcore_axis_name="s",
                                num_cores=NC, num_subcores=NS),
    scratch_types=[
        pltpu.VMEM((CH,), jnp.int32),
        pltpu.VMEM((TAIL,), jnp.int32),
        pltpu.VMEM((CH,), jnp.int32),
        pltpu.VMEM((SLICE,), jnp.int32),
        pltpu.VMEM((1024,), jnp.int32),
        pltpu.VMEM_SHARED((HPAD,), jnp.int32),
    ],
)


def _encode_body(x_ref, h_ref, zin_ref, zout_ref, o_ref):
    dout = jnp.minimum(h_ref[0, 0][:, :BLK], MAXDEG - 1)
    din = jnp.minimum(h_ref[1, 0][:, :BLK], MAXDEG - 1)
    iota = lax.broadcasted_iota(jnp.int32, (MAXDEG, BLK), 0)
    oh_in = (jnp.broadcast_to(din, (MAXDEG, BLK)) == iota).astype(jnp.float32)
    oh_out = (jnp.broadcast_to(dout, (MAXDEG, BLK)) == iota).astype(jnp.float32)
    dn = (((0,), (0,)), ((), ()))
    acc = x_ref[...]
    acc += lax.dot_general(oh_in, zin_ref[...], dn,
                           preferred_element_type=jnp.float32)
    acc += lax.dot_general(oh_out, zout_ref[...], dn,
                           preferred_element_type=jnp.float32)
    o_ref[...] = acc


_encode_call = pl.pallas_call(
    _encode_body,
    out_shape=jax.ShapeDtypeStruct((N, D), jnp.float32),
    grid=(NBLK,),
    in_specs=[
        pl.BlockSpec((BLK, D), lambda i: (i, 0)),
        pl.BlockSpec((NC, 1, 1, 1024), lambda i: (0, i, 0, 0)),
        pl.BlockSpec((MAXDEG, D), lambda i: (0, 0)),
        pl.BlockSpec((MAXDEG, D), lambda i: (0, 0)),
    ],
    out_specs=pl.BlockSpec((BLK, D), lambda i: (i, 0)),
)


def kernel(x, edge_index, z_in, z_out):
    e = edge_index.astype(jnp.int32)
    hist = _hist_call(e)
    return _encode_call(x, hist, z_in, z_out)


# speedup vs baseline: 3.0925x; 1.2173x over previous
import functools

import jax
import jax.numpy as jnp
from jax import lax
from jax.experimental import pallas as pl
from jax.experimental.pallas import tpu as pltpu
from jax.experimental.pallas import tpu_sc as plsc

MAXDEG = 64
N = 10000
E = 320000
D = 128
NC, NS = 2, 16
HPAD = 10240
SLICE = HPAD // NS
CH = 19968
TAIL = E - CH * NS
BLK = 2000
NBLK = N // BLK
HROW = 2048
UNR = 8


def _hist_body(edges, hist_out, idx_v, tail_v, ones_v, zeros_v, buf_v,
               hist_sh):
    c = lax.axis_index("c")
    s = lax.axis_index("s")

    one16 = jnp.ones((16,), jnp.int32)
    zero16 = jnp.zeros((16,), jnp.int32)

    def obody(i, carry):
        for u in range(UNR):
            ones_v[pl.ds(i * (16 * UNR) + u * 16, 16)] = one16
        return carry

    lax.fori_loop(0, CH // (16 * UNR), obody, 0)

    def zbody(i, carry):
        zeros_v[pl.ds(i * 16, 16)] = zero16
        return carry

    lax.fori_loop(0, SLICE // 16, zbody, 0)
    pltpu.sync_copy(zeros_v, hist_sh.at[pl.ds(s * SLICE, SLICE)])
    pltpu.sync_copy(edges.at[c, pl.ds(s * CH, CH)], idx_v)

    @pl.when(s == 0)
    def _():
        pltpu.sync_copy(edges.at[c, pl.ds(CH * NS, TAIL)], tail_v)

    plsc.subcore_barrier()
    pltpu.sync_copy(ones_v, hist_sh.at[idx_v], add=True)

    @pl.when(s == 0)
    def _():
        pltpu.sync_copy(ones_v.at[pl.ds(0, TAIL)], hist_sh.at[tail_v],
                        add=True)

    plsc.subcore_barrier()

    @pl.when(s < NBLK)
    def _():
        pltpu.sync_copy(hist_sh.at[pl.ds(s * BLK, HROW)], buf_v)
        pltpu.sync_copy(buf_v, hist_out.at[c, s, 0])


_hist_call = pl.kernel(
    _hist_body,
    out_type=jax.ShapeDtypeStruct((NC, NBLK, 1, HROW), jnp.int32),
    mesh=plsc.VectorSubcoreMesh(core_axis_name="c", subcore_axis_name="s",
                                num_cores=NC, num_subcores=NS),
    scratch_types=[
        pltpu.VMEM((CH,), jnp.int32),
        pltpu.VMEM((TAIL,), jnp.int32),
        pltpu.VMEM((CH,), jnp.int32),
        pltpu.VMEM((SLICE,), jnp.int32),
        pltpu.VMEM((HROW,), jnp.int32),
        pltpu.VMEM_SHARED((HPAD,), jnp.int32),
    ],
)


def _encode_body(x_ref, h_ref, ztab_ref, o_ref):
    dout = jnp.minimum(h_ref[0, 0][:, :BLK], MAXDEG - 1)
    din = jnp.minimum(h_ref[1, 0][:, :BLK], MAXDEG - 1)
    iota = lax.broadcasted_iota(jnp.int32, (2 * MAXDEG, BLK), 0)
    sel = ((iota == jnp.broadcast_to(dout, (2 * MAXDEG, BLK))) &
           (iota < MAXDEG)) | (
          iota == jnp.broadcast_to(din + MAXDEG, (2 * MAXDEG, BLK)))
    oh = sel.astype(jnp.float32)
    dn = (((0,), (0,)), ((), ()))
    o_ref[...] = x_ref[...] + lax.dot_general(
        oh, ztab_ref[...], dn, preferred_element_type=jnp.float32)


_encode_call = pl.pallas_call(
    _encode_body,
    out_shape=jax.ShapeDtypeStruct((N, D), jnp.float32),
    grid=(NBLK,),
    in_specs=[
        pl.BlockSpec((BLK, D), lambda i: (i, 0)),
        pl.BlockSpec((NC, 1, 1, HROW), lambda i: (0, i, 0, 0)),
        pl.BlockSpec((2 * MAXDEG, D), lambda i: (0, 0)),
    ],
    out_specs=pl.BlockSpec((BLK, D), lambda i: (i, 0)),
)


def kernel(x, edge_index, z_in, z_out):
    e = edge_index.astype(jnp.int32)
    hist = _hist_call(e)
    ztab = jnp.concatenate([z_out, z_in], axis=0)
    return _encode_call(x, hist, ztab)
